# baseline (speedup 1.0000x reference)
import functools

import jax
import jax.numpy as jnp
from jax import lax
from jax.experimental import pallas as pl
from jax.experimental.pallas import tpu as pltpu
from jax.experimental.pallas import tpu_sc as plsc

TEMP_VOCAB = 25
D = 256
ROWS = 4096 * 200
NUM_WORKERS = 32
PER_W = ROWS // NUM_WORKERS
B = 80
NBATCH = PER_W // B
NBUF = 4
LANES = 16
IDS = 256
ZOFF = 128


def _make_kernel():
    mesh = plsc.VectorSubcoreMesh(core_axis_name="c", subcore_axis_name="s")

    @functools.partial(
        pl.kernel,
        mesh=mesh,
        out_type=jax.ShapeDtypeStruct((ROWS, D), jnp.float32),
        scratch_types=[
            pltpu.VMEM((NBUF, IDS), jnp.int32),
            pltpu.VMEM((NBUF, B, D), jnp.float32),
            pltpu.VMEM((TEMP_VOCAB, D), jnp.float32),
        ] + [pltpu.SemaphoreType.DMA] * 12,
    )
    def k(idp_hbm, ttab_hbm, ztab_hbm, out_hbm,
          idb, zbuf, ttab_v, *sems):
        semi = sems[0:NBUF]
        semz = sems[NBUF:2 * NBUF]
        semo = sems[2 * NBUF:3 * NBUF]
        sid = lax.axis_index("s")
        wid = sid * 2 + lax.axis_index("c")
        base = wid * PER_W

        pltpu.sync_copy(ttab_hbm, ttab_v)

        def idoff(i):
            return (wid * NBATCH + i) * IDS

        def start_ids(i, b):
            pltpu.async_copy(idp_hbm.at[pl.ds(idoff(i), IDS)], idb.at[b], semi[b])

        def wait_ids(i, b):
            pltpu.make_async_copy(
                idp_hbm.at[pl.ds(idoff(i), IDS)], idb.at[b], semi[b]).wait()

        def start_gather(b):
            pltpu.async_copy(ztab_hbm.at[idb.at[b, pl.ds(ZOFF, B)]],
                             zbuf.at[b], semz[b])

        def wait_gather(b):
            pltpu.make_async_copy(ztab_hbm.at[idb.at[b, pl.ds(ZOFF, B)]],
                                  zbuf.at[b], semz[b]).wait()

        def outwrite(i, b):
            pltpu.async_copy(
                zbuf.at[b], out_hbm.at[pl.ds(base + i * B, B)], semo[b])

        def wait_out(i, b):
            pltpu.make_async_copy(
                zbuf.at[b], out_hbm.at[pl.ds(base + i * B, B)], semo[b]).wait()

        start_ids(0, 0)
        start_ids(1, 1)
        start_ids(2, 2)
        start_ids(3, 3)
        wait_ids(0, 0)
        start_gather(0)
        wait_ids(1, 1)
        start_gather(1)
        wait_ids(2, 2)
        start_gather(2)

        def ring_body(gi, carry):
            for bb in range(NBUF):
                i = gi * NBUF + bb
                b3 = (bb + 3) % NBUF

                @pl.when(i + 3 < NBATCH)
                def _():
                    wait_ids(i + 3, b3)

                    @pl.when(i >= 1)
                    def _():
                        wait_out(i - 1, b3)

                    start_gather(b3)

                wait_gather(bb)
                def group_body(g, c):
                    for q in range(LANES):
                        rr = g * LANES + q
                        for j in range(D // LANES):
                            sl = pl.ds(j * LANES, LANES)
                            zbuf[bb, rr, sl] = zbuf[bb, rr, sl] + ttab_v[0, sl]
                    return c

                lax.fori_loop(0, B // LANES, group_body, 0)

                @pl.when(i + 4 < NBATCH)
                def _():
                    start_ids(i + 4, bb)

                outwrite(i, bb)
            return carry

        lax.fori_loop(0, NBATCH // NBUF, ring_body, 0)
        for tail in range(NBATCH - 4, NBATCH):
            wait_out(tail, tail % NBUF)

    return k


_kernel = _make_kernel()


def kernel(temporal_id, zone_id, temporal_table, zone_table):
    tid = temporal_id.reshape(-1).astype(jnp.int32)
    zid = zone_id.reshape(-1).astype(jnp.int32)
    nblk = ROWS // B
    pad = jnp.zeros((nblk, ZOFF - B), jnp.int32)
    idpack = jnp.concatenate(
        [tid.reshape(nblk, B), pad, zid.reshape(nblk, B), pad],
        axis=1).reshape(-1)
    out = _kernel(idpack, temporal_table, zone_table)
    return out.reshape(temporal_id.shape + (D,))


# speedup vs baseline: 2.5127x; 2.5127x over previous
import functools

import jax
import jax.numpy as jnp
from jax import lax
from jax.experimental import pallas as pl
from jax.experimental.pallas import tpu as pltpu
from jax.experimental.pallas import tpu_sc as plsc

TEMP_VOCAB = 25
D = 256
ROWS = 4096 * 200
NUM_WORKERS = 32
PER_W = ROWS // NUM_WORKERS
B = 80
NBATCH = PER_W // B
NBUF = 4
LANES = 16
IDS = 256
ZOFF = 128


def _make_kernel():
    mesh = plsc.VectorSubcoreMesh(core_axis_name="c", subcore_axis_name="s")

    @functools.partial(
        pl.kernel,
        mesh=mesh,
        out_type=jax.ShapeDtypeStruct((ROWS, D), jnp.float32),
        scratch_types=[
            pltpu.VMEM((NBUF, IDS), jnp.int32),
            pltpu.VMEM((NBUF, B, D), jnp.float32),
            pltpu.VMEM((TEMP_VOCAB, D), jnp.float32),
        ] + [pltpu.SemaphoreType.DMA] * 12,
    )
    def k(idp_hbm, ttab_hbm, ztab_hbm, out_hbm,
          idb, zbuf, ttab_v, *sems):
        semi = sems[0:NBUF]
        semz = sems[NBUF:2 * NBUF]
        semo = sems[2 * NBUF:3 * NBUF]
        sid = lax.axis_index("s")
        wid = sid * 2 + lax.axis_index("c")
        base = wid * PER_W

        pltpu.sync_copy(ttab_hbm, ttab_v)

        def idoff(i):
            return (wid * NBATCH + i) * IDS

        def start_ids(i, b):
            pltpu.async_copy(idp_hbm.at[pl.ds(idoff(i), IDS)], idb.at[b], semi[b])

        def wait_ids(i, b):
            pltpu.make_async_copy(
                idp_hbm.at[pl.ds(idoff(i), IDS)], idb.at[b], semi[b]).wait()

        def start_gather(b):
            pltpu.async_copy(ztab_hbm.at[idb.at[b, pl.ds(ZOFF, B)]],
                             zbuf.at[b], semz[b])

        def wait_gather(b):
            pltpu.make_async_copy(ztab_hbm.at[idb.at[b, pl.ds(ZOFF, B)]],
                                  zbuf.at[b], semz[b]).wait()

        def outwrite(i, b):
            pltpu.async_copy(
                zbuf.at[b], out_hbm.at[pl.ds(base + i * B, B)], semo[b])

        def wait_out(i, b):
            pltpu.make_async_copy(
                zbuf.at[b], out_hbm.at[pl.ds(base + i * B, B)], semo[b]).wait()

        start_ids(0, 0)
        start_ids(1, 1)
        start_ids(2, 2)
        start_ids(3, 3)
        wait_ids(0, 0)
        start_gather(0)
        wait_ids(1, 1)
        start_gather(1)
        wait_ids(2, 2)
        start_gather(2)

        def ring_body(gi, carry):
            for bb in range(NBUF):
                i = gi * NBUF + bb
                b3 = (bb + 3) % NBUF

                @pl.when(i + 3 < NBATCH)
                def _():
                    wait_ids(i + 3, b3)

                    @pl.when(i >= 1)
                    def _():
                        wait_out(i - 1, b3)

                    start_gather(b3)

                wait_gather(bb)
                def group_body(g, c):
                    tvec = idb[bb, pl.ds(g * LANES, LANES)]
                    for q in range(LANES):
                        rr = g * LANES + q
                        tid = tvec[q]
                        vals = [ttab_v[tid, pl.ds(j * LANES, LANES)]
                                for j in range(D // LANES)]
                        for j in range(D // LANES):
                            plsc.addupdate(
                                zbuf.at[bb, rr, pl.ds(j * LANES, LANES)],
                                vals[j])
                    return c

                lax.fori_loop(0, B // LANES, group_body, 0)

                @pl.when(i + 4 < NBATCH)
                def _():
                    start_ids(i + 4, bb)

                outwrite(i, bb)
            return carry

        lax.fori_loop(0, NBATCH // NBUF, ring_body, 0)
        for tail in range(NBATCH - 4, NBATCH):
            wait_out(tail, tail % NBUF)

    return k


_kernel = _make_kernel()


def kernel(temporal_id, zone_id, temporal_table, zone_table):
    tid = temporal_id.reshape(-1).astype(jnp.int32)
    zid = zone_id.reshape(-1).astype(jnp.int32)
    nblk = ROWS // B
    pad = jnp.zeros((nblk, ZOFF - B), jnp.int32)
    idpack = jnp.concatenate(
        [tid.reshape(nblk, B), pad, zid.reshape(nblk, B), pad],
        axis=1).reshape(-1)
    out = _kernel(idpack, temporal_table, zone_table)
    return out.reshape(temporal_id.shape + (D,))


# speedup vs baseline: 2.5509x; 1.0152x over previous
import functools

import jax
import jax.numpy as jnp
from jax import lax
from jax.experimental import pallas as pl
from jax.experimental.pallas import tpu as pltpu
from jax.experimental.pallas import tpu_sc as plsc

TEMP_VOCAB = 25
D = 256
ROWS = 4096 * 200
NUM_WORKERS = 32
PER_W = ROWS // NUM_WORKERS
B = 80
NBATCH = PER_W // B
NBUF = 5
LANES = 16
IDS = 256
ZOFF = 128


def _make_kernel():
    mesh = plsc.VectorSubcoreMesh(core_axis_name="c", subcore_axis_name="s")

    @functools.partial(
        pl.kernel,
        mesh=mesh,
        out_type=jax.ShapeDtypeStruct((ROWS, D), jnp.float32),
        scratch_types=[
            pltpu.VMEM((NBUF, 1, IDS), jnp.int32),
            pltpu.VMEM((NBUF, B, D), jnp.float32),
            pltpu.VMEM((TEMP_VOCAB, D), jnp.float32),
        ] + [pltpu.SemaphoreType.DMA] * (3 * NBUF),
    )
    def k(idp_hbm, ttab_hbm, ztab_hbm, out_hbm,
          idb, zbuf, ttab_v, *sems):
        semi = sems[0:NBUF]
        semz = sems[NBUF:2 * NBUF]
        semo = sems[2 * NBUF:3 * NBUF]
        sid = lax.axis_index("s")
        wid = sid * 2 + lax.axis_index("c")
        base = wid * PER_W

        pltpu.sync_copy(ttab_hbm, ttab_v)

        def idoff(i):
            return wid * NBATCH + i

        def start_ids(i, b):
            pltpu.async_copy(idp_hbm.at[pl.ds(idoff(i), 1)], idb.at[b], semi[b])

        def wait_ids(i, b):
            pltpu.make_async_copy(
                idp_hbm.at[pl.ds(idoff(i), 1)], idb.at[b], semi[b]).wait()

        def start_gather(b):
            pltpu.async_copy(ztab_hbm.at[idb.at[b, 0, pl.ds(ZOFF, B)]],
                             zbuf.at[b], semz[b])

        def wait_gather(b):
            pltpu.make_async_copy(ztab_hbm.at[idb.at[b, 0, pl.ds(ZOFF, B)]],
                                  zbuf.at[b], semz[b]).wait()

        def outwrite(i, b):
            pltpu.async_copy(
                zbuf.at[b], out_hbm.at[pl.ds(base + i * B, B)], semo[b])

        def wait_out(i, b):
            pltpu.make_async_copy(
                zbuf.at[b], out_hbm.at[pl.ds(base + i * B, B)], semo[b]).wait()

        for b in range(NBUF):
            start_ids(b, b)
        for b in range(NBUF - 1):
            wait_ids(b, b)
            start_gather(b)

        def ring_body(gi, carry):
            for bb in range(NBUF):
                i = gi * NBUF + bb
                bn = (bb + NBUF - 1) % NBUF

                @pl.when(i + NBUF - 1 < NBATCH)
                def _():
                    wait_ids(i + NBUF - 1, bn)

                    @pl.when(i >= 1)
                    def _():
                        wait_out(i - 1, bn)

                    start_gather(bn)

                wait_gather(bb)
                def group_body(g, c):
                    tvec = idb[bb, 0, pl.ds(g * LANES, LANES)]
                    for q in range(LANES):
                        rr = g * LANES + q
                        tid = tvec[q]
                        vals = [ttab_v[tid, pl.ds(j * LANES, LANES)]
                                for j in range(D // LANES)]
                        for j in range(D // LANES):
                            plsc.addupdate(
                                zbuf.at[bb, rr, pl.ds(j * LANES, LANES)],
                                vals[j])
                    return c

                lax.fori_loop(0, B // LANES, group_body, 0)

                @pl.when(i + NBUF < NBATCH)
                def _():
                    start_ids(i + NBUF, bb)

                outwrite(i, bb)
            return carry

        lax.fori_loop(0, NBATCH // NBUF, ring_body, 0)
        for tail in range(NBATCH - NBUF, NBATCH):
            wait_out(tail, tail % NBUF)

    return k


_kernel = _make_kernel()


def kernel(temporal_id, zone_id, temporal_table, zone_table):
    tid = temporal_id.reshape(-1).astype(jnp.int32)
    zid = zone_id.reshape(-1).astype(jnp.int32)
    nblk = ROWS // B
    pad = jnp.zeros((nblk, ZOFF - B), jnp.int32)
    idpack = jnp.concatenate(
        [tid.reshape(nblk, B), pad, zid.reshape(nblk, B), pad], axis=1)
    out = _kernel(idpack, temporal_table, zone_table)
    return out.reshape(temporal_id.shape + (D,))
